# baseline (speedup 1.0000x reference)
import functools

import jax
import jax.numpy as jnp
from jax.experimental import pallas as pl
from jax.experimental.pallas import tpu as pltpu


def _fused_body(adj_hbm, seq_ref, nt_ref, wt_ref, gtab_ref, btab_ref,
                bias_ref, a_ref, out_ref, sf_ref, bufs, sems, *, bm, nbuf):
    i = pl.program_id(0)
    nsteps = pl.num_programs(0)

    h0 = (bm // 2 + 7) // 8 * 8
    halves = ((0, h0), (h0, bm - h0))

    def start_copy(step, slot):
        for h, (off, rows) in enumerate(halves):
            pltpu.make_async_copy(
                adj_hbm.at[pl.ds(step * bm + off, rows), :],
                bufs.at[slot, pl.ds(off, rows), :],
                sems.at[slot, h],
            ).start()

    def wait_copy(step, slot):
        for h, (off, rows) in enumerate(halves):
            pltpu.make_async_copy(
                adj_hbm.at[pl.ds(step * bm + off, rows), :],
                bufs.at[slot, pl.ds(off, rows), :],
                sems.at[slot, h],
            ).wait()

    @pl.when(i == 0)
    def _():
        for b in range(nbuf):
            start_copy(b, b)
        sf_ref[:, :] = jnp.dot(seq_ref[:, :], wt_ref[:, :],
                               preferred_element_type=jnp.float32)

    slot = jax.lax.rem(i, nbuf)
    wait_copy(i, slot)

    acc = jnp.dot(bufs[slot], sf_ref[:, :], preferred_element_type=jnp.float32)
    t = nt_ref[:, :]
    gamma = jnp.where(t == 0.0, gtab_ref[0:1, :], gtab_ref[1:2, :])
    beta = jnp.where(t == 0.0, btab_ref[0:1, :], btab_ref[1:2, :])
    o = gamma * acc + beta + bias_ref[:, :] + sf_ref[pl.ds(i * bm, bm), :]
    alpha = a_ref[0, 0]
    out_ref[:, :] = jnp.where(o >= 0.0, o, alpha * o)

    @pl.when(i + nbuf < nsteps)
    def _():
        start_copy(i + nbuf, slot)


def kernel(seq, adj, node_type, W, Wg, bg, Wb, bb, bias, a):
    n, d_in = seq.shape
    d_out = W.shape[0]

    wt = W.T
    gtab = Wg.T + bg[None, :]
    btab = Wb.T + bb[None, :]
    nt = node_type.astype(jnp.float32).reshape(n, 1)
    bias2 = bias.reshape(1, d_out)
    a2 = a.reshape(1, 1)

    bm = 200
    nbuf = 3
    out = pl.pallas_call(
        functools.partial(_fused_body, bm=bm, nbuf=nbuf),
        grid=(n // bm,),
        in_specs=[
            pl.BlockSpec(memory_space=pl.ANY),
            pl.BlockSpec((n, d_in), lambda i: (0, 0)),
            pl.BlockSpec((bm, 1), lambda i: (i, 0)),
            pl.BlockSpec((d_in, d_out), lambda i: (0, 0)),
            pl.BlockSpec((2, d_out), lambda i: (0, 0)),
            pl.BlockSpec((2, d_out), lambda i: (0, 0)),
            pl.BlockSpec((1, d_out), lambda i: (0, 0)),
            pl.BlockSpec((1, 1), lambda i: (0, 0)),
        ],
        out_specs=pl.BlockSpec((bm, d_out), lambda i: (i, 0)),
        out_shape=jax.ShapeDtypeStruct((n, d_out), jnp.float32),
        scratch_shapes=[
            pltpu.VMEM((n, d_out), jnp.float32),
            pltpu.VMEM((nbuf, bm, n), jnp.float32),
            pltpu.SemaphoreType.DMA((nbuf, 2)),
        ],
        compiler_params=pltpu.CompilerParams(
            dimension_semantics=("arbitrary",),
        ),
    )(adj, seq, nt, wt, gtab, btab, bias2, a2)
    return out


# speedup vs baseline: 1.0237x; 1.0237x over previous
import functools

import jax
import jax.numpy as jnp
from jax.experimental import pallas as pl
from jax.experimental.pallas import tpu as pltpu


def _fused_body(adj_hbm, seq_ref, nt_ref, wt_ref, gtab_ref, btab_ref,
                bias_ref, a_ref, out_ref, sf_ref, bufs, sems, *, bm, nbuf):
    i = pl.program_id(0)
    nsteps = pl.num_programs(0)

    def _copy(step, slot):
        return pltpu.make_async_copy(
            adj_hbm.at[pl.ds(step * bm, bm), :],
            bufs.at[slot],
            sems.at[slot],
        )

    @pl.when(i == 0)
    def _():
        for b in range(nbuf):
            _copy(b, b).start()
        sf_ref[:, :] = jnp.dot(seq_ref[:, :], wt_ref[:, :],
                               preferred_element_type=jnp.float32)

    nxt = i + nbuf - 1
    @pl.when(jnp.logical_and(i >= 1, nxt < nsteps))
    def _():
        _copy(nxt, jax.lax.rem(nxt, nbuf)).start()

    slot = jax.lax.rem(i, nbuf)
    _copy(i, slot).wait()

    acc = jnp.dot(bufs[slot], sf_ref[:, :], preferred_element_type=jnp.float32)
    t = nt_ref[:, :]
    gamma = jnp.where(t == 0.0, gtab_ref[0:1, :], gtab_ref[1:2, :])
    beta = jnp.where(t == 0.0, btab_ref[0:1, :], btab_ref[1:2, :])
    o = gamma * acc + beta + bias_ref[:, :] + sf_ref[pl.ds(i * bm, bm), :]
    alpha = a_ref[0, 0]
    out_ref[:, :] = jnp.where(o >= 0.0, o, alpha * o)


def kernel(seq, adj, node_type, W, Wg, bg, Wb, bb, bias, a):
    n, d_in = seq.shape
    d_out = W.shape[0]

    wt = W.T
    gtab = Wg.T + bg[None, :]
    btab = Wb.T + bb[None, :]
    nt = node_type.astype(jnp.float32).reshape(n, 1)
    bias2 = bias.reshape(1, d_out)
    a2 = a.reshape(1, 1)

    bm = 200
    nbuf = 3
    out = pl.pallas_call(
        functools.partial(_fused_body, bm=bm, nbuf=nbuf),
        grid=(n // bm,),
        in_specs=[
            pl.BlockSpec(memory_space=pl.ANY),
            pl.BlockSpec((n, d_in), lambda i: (0, 0)),
            pl.BlockSpec((bm, 1), lambda i: (i, 0)),
            pl.BlockSpec((d_in, d_out), lambda i: (0, 0)),
            pl.BlockSpec((2, d_out), lambda i: (0, 0)),
            pl.BlockSpec((2, d_out), lambda i: (0, 0)),
            pl.BlockSpec((1, d_out), lambda i: (0, 0)),
            pl.BlockSpec((1, 1), lambda i: (0, 0)),
        ],
        out_specs=pl.BlockSpec((bm, d_out), lambda i: (i, 0)),
        out_shape=jax.ShapeDtypeStruct((n, d_out), jnp.float32),
        scratch_shapes=[
            pltpu.VMEM((n, d_out), jnp.float32),
            pltpu.VMEM((nbuf, bm, n), jnp.float32),
            pltpu.SemaphoreType.DMA((nbuf,)),
        ],
        compiler_params=pltpu.CompilerParams(
            dimension_semantics=("arbitrary",),
        ),
    )(adj, seq, nt, wt, gtab, btab, bias2, a2)
    return out
